# baseline (speedup 1.0000x reference)
import functools

import jax
import jax.numpy as jnp
from jax import lax
from jax.experimental import pallas as pl
from jax.experimental.pallas import tpu as pltpu
from jax.experimental.pallas import tpu_sc as plsc

_M, _N = 8, 4
_SH, _RH = 1024, 256
_T = 512
_LO, _HI = -10.0, 10.0
_H = (_HI - _LO) / (_T - 1)
_INVH = 1.0 / _H


def _build_kernel(dt_ref, g_ref, a_ref, b_ref, ws1_ref, bs1_ref, ws2_ref,
                  bs2_ref, w1r_ref, b1r_ref, w2r_ref, b2r_ref,
                  tab_ref, c_ref):
    dt = dt_ref[0, 0]
    g = g_ref[...]
    s = 0.5 * (g + jnp.swapaxes(g, 1, 2))
    w = jax.nn.sigmoid(s)
    ii = lax.broadcasted_iota(jnp.int32, (_M, _N, _N), 1)
    jj = lax.broadcasted_iota(jnp.int32, (_M, _N, _N), 2)
    eye = (ii == jj)
    w = jnp.where(eye, 0.0, w)
    deg = jnp.sum(w, axis=2, keepdims=True)
    for m in range(_M):
        c_ref[m:m + 1, :, :] = dt * (
            a_ref[m] * jnp.where(eye[m:m + 1], deg[m:m + 1], 0.0)
            + b_ref[m] * w[m:m + 1])

    xs = lax.broadcasted_iota(jnp.int32, (_T, 1), 0).astype(jnp.float32) * _H + _LO
    t = jnp.tanh(xs * ws1_ref[...] + bs1_ref[...])
    stab = jnp.sum(t * ws2_ref[...], axis=1) + bs2_ref[0, 0]
    for m in range(_M):
        tt = jnp.tanh(xs * w1r_ref[m, :] + b1r_ref[m, :])
        colm = jnp.sum(tt * w2r_ref[m, :], axis=1)
        tab_ref[pl.ds(m * _T, _T)] = dt * (stab + colm + b2r_ref[m])


def kernel(X, z, dt_val, graph_logits, alpha, beta, ws1, bs1, ws2, bs2,
           w1r, b1r, w2r, b2r):
    B = X.shape[0]
    f32 = jnp.float32

    dt = jnp.asarray(dt_val, f32).reshape(1, 1)
    smem = pl.BlockSpec(memory_space=pltpu.SMEM)
    full = lambda shape: pl.BlockSpec(shape, lambda: (0,) * len(shape))
    tab, ctab = pl.pallas_call(
        _build_kernel,
        in_specs=[
            smem,
            full((_M, _N, _N)),
            smem, smem,
            full((_SH,)), full((_SH,)),
            full((_SH,)), smem,
            full((_M, _RH)), full((_M, _RH)),
            full((_M, _RH)), smem,
        ],
        out_shape=[
            jax.ShapeDtypeStruct((_M * _T,), f32),
            jax.ShapeDtypeStruct((_M, _N, _N), f32),
        ],
    )(dt, graph_logits, alpha, beta, ws1, bs1, ws2, bs2.reshape(1, 1),
      w1r, b1r, w2r, b2r)

    zi = z.astype(jnp.int32)

    mesh = plsc.VectorSubcoreMesh(core_axis_name="c", subcore_axis_name="s")
    nw = mesh.num_cores * mesh.num_subcores
    spw = (B * _N) // nw
    tpw = B // nw

    @functools.partial(
        pl.kernel,
        out_type=jax.ShapeDtypeStruct((B, _N), f32),
        mesh=mesh,
        scratch_types=[
            pltpu.VMEM((tpw, _N), f32),
            pltpu.VMEM((tpw,), jnp.int32),
            pltpu.VMEM((_M * _T,), f32),
            pltpu.VMEM((_M, _N, _N), f32),
            pltpu.VMEM((tpw, _N), f32),
        ],
        compiler_params=pltpu.CompilerParams(needs_layout_passes=False),
    )
    def _sc_lookup(x_hbm, z_hbm, tab_hbm, c_hbm, out_hbm, xv, zv, rv, cv, ov):
        wid = lax.axis_index("s") * mesh.num_cores + lax.axis_index("c")
        tbase = wid * tpw
        pltpu.sync_copy(x_hbm.at[pl.ds(tbase, tpw), :], xv)
        pltpu.sync_copy(z_hbm.at[pl.ds(tbase, tpw)], zv)
        pltpu.sync_copy(tab_hbm, rv)
        pltpu.sync_copy(c_hbm, cv)
        lane = lax.iota(jnp.int32, 16)
        n = lax.bitwise_and(lane, 3)
        tokoff = lax.shift_right_logical(lane, 2)
        perm = [lane - n + j for j in range(_N)]

        def body(gi, carry):
            for u in range(4):
                it = gi * 4 + u
                t0 = it * 4
                tok = t0 + tokoff
                x = plsc.load_gather(xv, [tok, n])
                zt = plsc.load_gather(zv, [tok])
                uu = (x - _LO) * _INVH
                uu = jnp.minimum(jnp.maximum(uu, 0.0), _T - 1.001)
                ti = uu.astype(jnp.int32)
                fr = uu - ti.astype(f32)
                gi0 = zt * _T + ti
                r0 = plsc.load_gather(rv, [gi0])
                r1 = plsc.load_gather(rv, [gi0 + 1])
                acc = r0 + fr * (r1 - r0)
                for j in range(_N):
                    cj = plsc.load_gather(
                        cv, [zt, n, jnp.full((16,), j, jnp.int32)])
                    xj = plsc.load_gather(
                        xv, [tok, jnp.full((16,), j, jnp.int32)])
                    acc = acc + cj * xj
                plsc.store_scatter(ov, [tok, n], x + acc)
            return carry

        lax.fori_loop(0, spw // 64, body, 0)
        pltpu.sync_copy(ov, out_hbm.at[pl.ds(tbase, tpw), :])

    return _sc_lookup(X, zi, tab, ctab)


# speedup vs baseline: 1.3488x; 1.3488x over previous
import functools

import jax
import jax.numpy as jnp
from jax import lax
from jax.experimental import pallas as pl
from jax.experimental.pallas import tpu as pltpu
from jax.experimental.pallas import tpu_sc as plsc

_M, _N = 8, 4
_SH, _RH = 1024, 256
_T = 512
_LO, _HI = -10.0, 10.0
_H = (_HI - _LO) / (_T - 1)
_INVH = 1.0 / _H


def _build_kernel(dt_ref, g_ref, a_ref, b_ref, ws1_ref, bs1_ref, ws2_ref,
                  bs2_ref, w1r_ref, b1r_ref, w2r_ref, b2r_ref,
                  tab_ref, c_ref):
    dt = dt_ref[0, 0]
    g = g_ref[...]
    s = 0.5 * (g + jnp.swapaxes(g, 1, 2))
    w = jax.nn.sigmoid(s)
    ii = lax.broadcasted_iota(jnp.int32, (_M, _N, _N), 1)
    jj = lax.broadcasted_iota(jnp.int32, (_M, _N, _N), 2)
    eye = (ii == jj)
    w = jnp.where(eye, 0.0, w)
    deg = jnp.sum(w, axis=2, keepdims=True)
    for m in range(_M):
        c_ref[m:m + 1, :, :] = dt * (
            a_ref[m] * jnp.where(eye[m:m + 1], deg[m:m + 1], 0.0)
            + b_ref[m] * w[m:m + 1])

    xs = lax.broadcasted_iota(jnp.int32, (_T, 1), 0).astype(jnp.float32) * _H + _LO
    t = jnp.tanh(xs * ws1_ref[...] + bs1_ref[...])
    stab = jnp.sum(t * ws2_ref[...], axis=1) + bs2_ref[0, 0]
    for m in range(_M):
        tt = jnp.tanh(xs * w1r_ref[m, :] + b1r_ref[m, :])
        colm = jnp.sum(tt * w2r_ref[m, :], axis=1)
        tab_ref[pl.ds(m * _T, _T)] = dt * (stab + colm + b2r_ref[m])


def kernel(X, z, dt_val, graph_logits, alpha, beta, ws1, bs1, ws2, bs2,
           w1r, b1r, w2r, b2r):
    B = X.shape[0]
    f32 = jnp.float32

    dt = jnp.asarray(dt_val, f32).reshape(1, 1)
    smem = pl.BlockSpec(memory_space=pltpu.SMEM)
    full = lambda shape: pl.BlockSpec(shape, lambda: (0,) * len(shape))
    tab, ctab = pl.pallas_call(
        _build_kernel,
        in_specs=[
            smem,
            full((_M, _N, _N)),
            smem, smem,
            full((_SH,)), full((_SH,)),
            full((_SH,)), smem,
            full((_M, _RH)), full((_M, _RH)),
            full((_M, _RH)), smem,
        ],
        out_shape=[
            jax.ShapeDtypeStruct((_M * _T,), f32),
            jax.ShapeDtypeStruct((_M, _N, _N), f32),
        ],
    )(dt, graph_logits, alpha, beta, ws1, bs1, ws2, bs2.reshape(1, 1),
      w1r, b1r, w2r, b2r)

    xT = X.T
    zi = z.astype(jnp.int32)
    ctabf = ctab.reshape(_M * _N * _N)

    mesh = plsc.VectorSubcoreMesh(core_axis_name="c", subcore_axis_name="s")
    nw = mesh.num_cores * mesh.num_subcores
    tpw = B // nw

    @functools.partial(
        pl.kernel,
        out_type=jax.ShapeDtypeStruct((_N, B), f32),
        mesh=mesh,
        scratch_types=[
            [pltpu.VMEM((tpw,), f32) for _ in range(_N)],
            pltpu.VMEM((tpw,), jnp.int32),
            pltpu.VMEM((_M * _T,), f32),
            pltpu.VMEM((_M * _N * _N,), f32),
            [pltpu.VMEM((tpw,), f32) for _ in range(_N)],
        ],
        compiler_params=pltpu.CompilerParams(needs_layout_passes=False),
    )
    def _sc_lookup(x_hbm, z_hbm, tab_hbm, c_hbm, out_hbm,
                   xvs, zv, rv, cv, ovs):
        wid = lax.axis_index("s") * mesh.num_cores + lax.axis_index("c")
        tbase = wid * tpw
        for n in range(_N):
            pltpu.sync_copy(x_hbm.at[n, pl.ds(tbase, tpw)], xvs[n])
        pltpu.sync_copy(z_hbm.at[pl.ds(tbase, tpw)], zv)
        pltpu.sync_copy(tab_hbm, rv)
        pltpu.sync_copy(c_hbm, cv)

        def body(it, carry):
            t0 = it * 16
            zt = zv[pl.ds(t0, 16)]
            zt16 = zt * 16
            ztT = zt * _T
            xs_ = [xvs[j][pl.ds(t0, 16)] for j in range(_N)]
            for n in range(_N):
                x = xs_[n]
                uu = (x - _LO) * _INVH
                uu = jnp.minimum(jnp.maximum(uu, 0.0), _T - 1.001)
                ti = uu.astype(jnp.int32)
                fr = uu - ti.astype(f32)
                gi0 = ztT + ti
                r0 = plsc.load_gather(rv, [gi0])
                r1 = plsc.load_gather(rv, [gi0 + 1])
                acc = r0 + fr * (r1 - r0)
                for j in range(_N):
                    cj = plsc.load_gather(cv, [zt16 + (4 * n + j)])
                    acc = acc + cj * xs_[j]
                ovs[n][pl.ds(t0, 16)] = x + acc
            return carry

        lax.fori_loop(0, tpw // 16, body, 0)
        for n in range(_N):
            pltpu.sync_copy(ovs[n], out_hbm.at[n, pl.ds(tbase, tpw)])

    outT = _sc_lookup(xT, zi, tab, ctabf)
    return outT.T
